# baseline (speedup 1.0000x reference)
import functools

import jax
import jax.numpy as jnp
from jax.experimental import pallas as pl
from jax.experimental.pallas import tpu as pltpu

_SPARSITY = 0.15
_GAIN = 3.0


def _gated_topk_block(x_ref, o_ref, *, k):
    xb = x_ref[...]
    R, N = xb.shape

    bits = jax.lax.bitcast_convert_type(xb, jnp.uint32)
    sign = bits >= jnp.uint32(0x80000000)
    u = jnp.where(sign, ~bits, bits | jnp.uint32(0x80000000))

    thresh = jnp.zeros((R, 1), dtype=jnp.uint32)
    for bit in range(31, -1, -1):
        cand = thresh | jnp.uint32(1 << bit)
        cnt = jnp.sum((u >= cand).astype(jnp.int32), axis=1, keepdims=True)
        thresh = jnp.where(cnt >= k, cand, thresh)
    mask = u >= thresh

    m1 = jnp.max(xb, axis=1, keepdims=True)
    is_max = xb == m1
    nmax = jnp.sum(is_max.astype(jnp.int32), axis=1, keepdims=True)
    runner = jnp.max(jnp.where(is_max, -jnp.inf, xb), axis=1, keepdims=True)
    m2 = jnp.where(nmax >= 2, m1, runner)
    gain = jax.nn.sigmoid(m1 - m2) * _GAIN + 1.0

    o_ref[...] = jnp.where(mask, xb * gain, 0.0)


@jax.jit
def kernel(x):
    B, N = x.shape
    k = max(int(N * _SPARSITY), 2)
    R = 16
    grid = (B // R,)
    return pl.pallas_call(
        functools.partial(_gated_topk_block, k=k),
        grid=grid,
        in_specs=[pl.BlockSpec((R, N), lambda i: (i, 0))],
        out_specs=pl.BlockSpec((R, N), lambda i: (i, 0)),
        out_shape=jax.ShapeDtypeStruct((B, N), x.dtype),
        compiler_params=pltpu.CompilerParams(
            dimension_semantics=("parallel",),
        ),
    )(x)


# speedup vs baseline: 2.1140x; 2.1140x over previous
import functools

import jax
import jax.numpy as jnp
from jax.experimental import pallas as pl
from jax.experimental.pallas import tpu as pltpu

_SPARSITY = 0.15
_GAIN = 3.0
_CHUNK = 512
_RG = 8
_G = 4


def _gated_topk_block(x_ref, o_ref, hi0, hi1, hi2, hi3, lo0, lo1, lo2, lo3,
                      *, k):
    R, N = x_ref.shape
    nchunk = N // _CHUNK
    his, los = (hi0, hi1, hi2, hi3), (lo0, lo1, lo2, lo3)

    def sl(j):
        return slice(j * _CHUNK, (j + 1) * _CHUNK)

    def rows(g):
        return slice(g * _RG, (g + 1) * _RG)

    a = [[jnp.full((_RG, _CHUNK), -jnp.inf, jnp.float32) for _ in range(2)]
         for _ in range(_G)]
    b = [[jnp.full((_RG, _CHUNK), -jnp.inf, jnp.float32) for _ in range(2)]
         for _ in range(_G)]
    for j in range(nchunk):
        s = j % 2
        for g in range(_G):
            xc = x_ref[rows(g), sl(j)]
            bc = jax.lax.bitcast_convert_type(xc, jnp.uint32)
            h = bc >> 16
            neg = h >= jnp.uint32(0x8000)
            his[g][:, sl(j)] = (jnp.where(neg, h ^ jnp.uint32(0x7FFF), h)
                                .astype(jnp.int16))
            los[g][:, sl(j)] = ((jnp.where(neg, ~bc, bc)
                                 ^ jnp.uint32(0x8000)).astype(jnp.int16))
            b[g][s] = jnp.maximum(b[g][s], jnp.minimum(a[g][s], xc))
            a[g][s] = jnp.maximum(a[g][s], xc)

    def count_ge(a_ref, cand_row):
        cand16 = cand_row.astype(jnp.int16)
        acc = [jnp.zeros((_RG, _CHUNK), jnp.int16) for _ in range(2)]
        for j in range(nchunk):
            c = a_ref[:, sl(j)]
            acc[j % 2] = acc[j % 2] + jnp.where(
                c >= cand16, jnp.int16(1), jnp.int16(0))
        tot = acc[0] + acc[1]
        return jnp.sum(tot.astype(jnp.int32), axis=1, keepdims=True)

    def search(refs):
        p = [jnp.zeros((_RG, 1), dtype=jnp.int32) for _ in range(_G)]
        for bit in range(15, -1, -1):
            for g in range(_G):
                cand = p[g] | (1 << bit)
                cnt = count_ge(refs[g], cand ^ 0x8000)
                p[g] = jnp.where(cnt >= k, cand, p[g])
        return p

    p = search(his)

    for g in range(_G):
        p16 = (p[g] ^ 0x8000).astype(jnp.int16)
        for j in range(nchunk):
            hc = his[g][:, sl(j)]
            lc = los[g][:, sl(j)]
            los[g][:, sl(j)] = jnp.where(
                hc > p16, jnp.int16(32767),
                jnp.where(hc == p16, lc, jnp.int16(-32768)))

    q = search(los)

    for g in range(_G):
        key = (p[g].astype(jnp.uint32) << 16) | q[g].astype(jnp.uint32)
        tbits = jnp.where(key >= jnp.uint32(0x80000000),
                          key ^ jnp.uint32(0x80000000), ~key)
        tf = jax.lax.bitcast_convert_type(tbits, jnp.float32)

        am = jnp.maximum(a[g][0], a[g][1])
        bm = jnp.maximum(jnp.minimum(a[g][0], a[g][1]),
                         jnp.maximum(b[g][0], b[g][1]))
        m1 = jnp.max(am, axis=1, keepdims=True)
        amax = am == m1
        nmax = jnp.sum(amax.astype(jnp.int32), axis=1, keepdims=True)
        runner = jnp.max(jnp.where(amax, -jnp.inf, am), axis=1,
                         keepdims=True)
        b_at = jnp.max(jnp.where(amax, bm, -jnp.inf), axis=1, keepdims=True)
        m2 = jnp.where(nmax >= 2, m1, jnp.maximum(runner, b_at))
        gain = jax.nn.sigmoid(m1 - m2) * _GAIN + 1.0

        for j in range(nchunk):
            xc = x_ref[rows(g), sl(j)]
            o_ref[rows(g), sl(j)] = jnp.where(xc >= tf, xc * gain, 0.0)


@jax.jit
def kernel(x):
    B, N = x.shape
    k = max(int(N * _SPARSITY), 2)
    R = _RG * _G
    grid = (B // R,)
    return pl.pallas_call(
        functools.partial(_gated_topk_block, k=k),
        grid=grid,
        in_specs=[pl.BlockSpec((R, N), lambda i: (i, 0))],
        out_specs=pl.BlockSpec((R, N), lambda i: (i, 0)),
        out_shape=jax.ShapeDtypeStruct((B, N), x.dtype),
        scratch_shapes=[pltpu.VMEM((_RG, N), jnp.int16)] * (2 * _G),
        compiler_params=pltpu.CompilerParams(
            dimension_semantics=("parallel",),
        ),
    )(x)


# speedup vs baseline: 2.1336x; 1.0093x over previous
import functools

import jax
import jax.numpy as jnp
from jax.experimental import pallas as pl
from jax.experimental.pallas import tpu as pltpu

_SPARSITY = 0.15
_GAIN = 3.0
_CHUNK = 512
_RG = 8
_G = 4


def _gated_topk_block(x_ref, o_ref, hi0, hi1, hi2, hi3, lo0, lo1, lo2, lo3,
                      *, k):
    R, N = x_ref.shape
    nchunk = N // _CHUNK
    his, los = (hi0, hi1, hi2, hi3), (lo0, lo1, lo2, lo3)

    def sl(j):
        return slice(j * _CHUNK, (j + 1) * _CHUNK)

    def rows(g):
        return slice(g * _RG, (g + 1) * _RG)

    a = [[jnp.full((_RG, _CHUNK), -jnp.inf, jnp.float32) for _ in range(2)]
         for _ in range(_G)]
    b = [[jnp.full((_RG, _CHUNK), -jnp.inf, jnp.float32) for _ in range(2)]
         for _ in range(_G)]
    for j in range(nchunk):
        s = j % 2
        for g in range(_G):
            xc = x_ref[rows(g), sl(j)]
            bc = jax.lax.bitcast_convert_type(xc, jnp.uint32)
            neg = bc >= jnp.uint32(0x80000000)
            u = jnp.where(neg, ~bc, bc | jnp.uint32(0x80000000))
            his[g][:, sl(j)] = ((u >> 16) ^ jnp.uint32(0x8000)).astype(
                jnp.int16)
            los[g][:, sl(j)] = (u ^ jnp.uint32(0x8000)).astype(jnp.int16)
            b[g][s] = jnp.maximum(b[g][s], jnp.minimum(a[g][s], xc))
            a[g][s] = jnp.maximum(a[g][s], xc)

    def count_ge(a_ref, cand_row):
        cand16 = cand_row.astype(jnp.int16)
        acc = [jnp.zeros((_RG, _CHUNK), jnp.int16) for _ in range(2)]
        for j in range(nchunk):
            c = a_ref[:, sl(j)]
            acc[j % 2] = acc[j % 2] + jnp.where(
                c >= cand16, jnp.int16(1), jnp.int16(0))
        tot = acc[0] + acc[1]
        return jnp.sum(tot.astype(jnp.int32), axis=1, keepdims=True)

    def search(refs):
        p = [jnp.zeros((_RG, 1), dtype=jnp.int32) for _ in range(_G)]
        for bit in range(15, -1, -1):
            for g in range(_G):
                cand = p[g] | (1 << bit)
                cnt = count_ge(refs[g], cand ^ 0x8000)
                p[g] = jnp.where(cnt >= k, cand, p[g])
        return p

    p = search(his)

    for g in range(_G):
        p16 = (p[g] ^ 0x8000).astype(jnp.int16)
        for j in range(nchunk):
            hc = his[g][:, sl(j)]
            lc = los[g][:, sl(j)]
            los[g][:, sl(j)] = jnp.where(
                hc > p16, jnp.int16(32767),
                jnp.where(hc == p16, lc, jnp.int16(-32768)))

    q = search(los)

    for g in range(_G):
        key = (p[g].astype(jnp.uint32) << 16) | q[g].astype(jnp.uint32)
        tbits = jnp.where(key >= jnp.uint32(0x80000000),
                          key ^ jnp.uint32(0x80000000), ~key)
        tf = jax.lax.bitcast_convert_type(tbits, jnp.float32)

        am = jnp.maximum(a[g][0], a[g][1])
        bm = jnp.maximum(jnp.minimum(a[g][0], a[g][1]),
                         jnp.maximum(b[g][0], b[g][1]))
        m1 = jnp.max(am, axis=1, keepdims=True)
        amax = am == m1
        nmax = jnp.sum(amax.astype(jnp.int32), axis=1, keepdims=True)
        runner = jnp.max(jnp.where(amax, -jnp.inf, am), axis=1,
                         keepdims=True)
        b_at = jnp.max(jnp.where(amax, bm, -jnp.inf), axis=1, keepdims=True)
        m2 = jnp.where(nmax >= 2, m1, jnp.maximum(runner, b_at))
        gain = jax.nn.sigmoid(m1 - m2) * _GAIN + 1.0

        for j in range(nchunk):
            xc = x_ref[rows(g), sl(j)]
            o_ref[rows(g), sl(j)] = jnp.where(xc >= tf, xc * gain, 0.0)


@jax.jit
def kernel(x):
    B, N = x.shape
    k = max(int(N * _SPARSITY), 2)
    R = _RG * _G
    grid = (B // R,)
    return pl.pallas_call(
        functools.partial(_gated_topk_block, k=k),
        grid=grid,
        in_specs=[pl.BlockSpec((R, N), lambda i: (i, 0))],
        out_specs=pl.BlockSpec((R, N), lambda i: (i, 0)),
        out_shape=jax.ShapeDtypeStruct((B, N), x.dtype),
        scratch_shapes=[pltpu.VMEM((_RG, N), jnp.int16)] * (2 * _G),
        compiler_params=pltpu.CompilerParams(
            dimension_semantics=("parallel",),
        ),
    )(x)
